# baseline (speedup 1.0000x reference)
import functools

import jax
import jax.numpy as jnp
from jax import lax
from jax.experimental import pallas as pl
from jax.experimental.pallas import tpu as pltpu
from jax.experimental.pallas import tpu_sc as plsc

B, L, H = 4, 2048, 1024
N = B * L
NC, NS = 2, 16
NW = NC * NS
P = L // NW
PC = 8
NCH = P // PC
LANES = 16


def kernel(X, W_word, W_pos):
    idx = X.reshape(N).astype(jnp.int32)
    mesh = plsc.VectorSubcoreMesh(core_axis_name="c", subcore_axis_name="s")

    @functools.partial(
        pl.kernel,
        out_type=jax.ShapeDtypeStruct((N, H), jnp.float32),
        mesh=mesh,
        scratch_types=[
            pltpu.VMEM((B * P,), jnp.int32),
            pltpu.VMEM((B * PC, H), jnp.float32),
            pltpu.VMEM((B * PC, H), jnp.float32),
            pltpu.VMEM((B * PC, H), jnp.float32),
            pltpu.VMEM((PC, H), jnp.float32),
            pltpu.VMEM((PC, H), jnp.float32),
            pltpu.SemaphoreType.DMA,
            pltpu.SemaphoreType.DMA,
            pltpu.SemaphoreType.DMA,
            pltpu.SemaphoreType.DMA,
            pltpu.SemaphoreType.DMA,
            pltpu.SemaphoreType.DMA,
            pltpu.SemaphoreType.DMA,
            pltpu.SemaphoreType.DMA,
        ],
    )
    def embed(w_hbm, p_hbm, i_hbm, o_hbm,
              idx_v, ring0, ring1, ring2, pos0, pos1,
              sg0, sg1, sg2, so0, so1, so2, sp0, sp1):
        ring = [ring0, ring1, ring2]
        pos = [pos0, pos1]
        sg = [sg0, sg1, sg2]
        so = [so0, so1, so2]
        sp = [sp0, sp1]

        wid = lax.axis_index("s") * NC + lax.axis_index("c")
        l0 = wid * P

        for b in range(B):
            pltpu.sync_copy(i_hbm.at[pl.ds(b * L + l0, P)],
                            idx_v.at[pl.ds(b * P, P)])

        def start(k):
            p = k % 3
            gathers = [
                pltpu.async_copy(
                    w_hbm.at[idx_v.at[pl.ds(b * P + k * PC, PC)]],
                    ring[p].at[pl.ds(b * PC, PC)],
                    sg[p])
                for b in range(B)
            ]
            pcp = pltpu.async_copy(
                p_hbm.at[pl.ds(l0 + k * PC, PC)], pos[k % 2], sp[k % 2])
            return gathers, pcp

        inflight = {0: start(0), 1: start(1)}
        out_cp = {}

        for k in range(NCH):
            p = k % 3
            gathers, pcp = inflight.pop(k)
            for g in gathers:
                g.wait()
            pcp.wait()

            @pl.loop(0, PC)
            def _(r):
                @pl.loop(0, H, step=LANES)
                def _(c):
                    s = pl.ds(c, LANES)
                    pv = pos[k % 2].at[r, s][...]
                    for b in range(B):
                        ring[p].at[b * PC + r, s][...] = jnp.maximum(
                            ring[p].at[b * PC + r, s][...] + pv, 0.0)

            for b in range(B):
                out_cp[(k, b)] = pltpu.async_copy(
                    ring[p].at[pl.ds(b * PC, PC)],
                    o_hbm.at[pl.ds(b * L + l0 + k * PC, PC)],
                    so[p])

            if k + 2 < NCH:
                nxt = (k + 2) % 3
                if k - 1 >= 0:
                    for b in range(B):
                        out_cp.pop((k - 1, b)).wait()
                inflight[k + 2] = start(k + 2)

        for kk in (NCH - 2, NCH - 1):
            for b in range(B):
                out_cp.pop((kk, b)).wait()

    out = embed(W_word, W_pos, idx)
    return out.reshape(B, L, H)


# speedup vs baseline: 2.1799x; 2.1799x over previous
import functools

import jax
import jax.numpy as jnp
from jax import lax
from jax.experimental import pallas as pl
from jax.experimental.pallas import tpu as pltpu
from jax.experimental.pallas import tpu_sc as plsc

B, L, H = 4, 2048, 1024
N = B * L
NC, NS = 2, 16
NW = NC * NS
P = L // NW
PC = 8
NCH = P // PC
LANES = 16


def kernel(X, W_word, W_pos):
    idx = X.reshape(N).astype(jnp.int32)
    mesh = plsc.VectorSubcoreMesh(core_axis_name="c", subcore_axis_name="s")

    @functools.partial(
        pl.kernel,
        out_type=jax.ShapeDtypeStruct((N, H), jnp.float32),
        mesh=mesh,
        scratch_types=[
            pltpu.VMEM((B * P,), jnp.int32),
            pltpu.VMEM((B * PC, H), jnp.float32),
            pltpu.VMEM((B * PC, H), jnp.float32),
            pltpu.VMEM((B * PC, H), jnp.float32),
            pltpu.VMEM((PC, H), jnp.float32),
            pltpu.VMEM((PC, H), jnp.float32),
            pltpu.SemaphoreType.DMA,
            pltpu.SemaphoreType.DMA,
            pltpu.SemaphoreType.DMA,
            pltpu.SemaphoreType.DMA,
            pltpu.SemaphoreType.DMA,
            pltpu.SemaphoreType.DMA,
            pltpu.SemaphoreType.DMA,
            pltpu.SemaphoreType.DMA,
        ],
    )
    def embed(w_hbm, p_hbm, i_hbm, o_hbm,
              idx_v, ring0, ring1, ring2, pos0, pos1,
              sg0, sg1, sg2, so0, so1, so2, sp0, sp1):
        ring = [ring0, ring1, ring2]
        pos = [pos0, pos1]
        sg = [sg0, sg1, sg2]
        so = [so0, so1, so2]
        sp = [sp0, sp1]

        wid = lax.axis_index("s") * NC + lax.axis_index("c")
        l0 = wid * P

        for b in range(B):
            pltpu.sync_copy(i_hbm.at[pl.ds(b * L + l0, P)],
                            idx_v.at[pl.ds(b * P, P)])

        def start(k):
            p = k % 3
            gathers = [
                pltpu.async_copy(
                    w_hbm.at[idx_v.at[pl.ds(b * P + k * PC, PC)]],
                    ring[p].at[pl.ds(b * PC, PC)],
                    sg[p])
                for b in range(B)
            ]
            pcp = pltpu.async_copy(
                p_hbm.at[pl.ds(l0 + k * PC, PC)], pos[k % 2], sp[k % 2])
            return gathers, pcp

        inflight = {0: start(0), 1: start(1)}
        out_cp = {}

        for k in range(NCH):
            p = k % 3
            gathers, pcp = inflight.pop(k)
            for g in gathers:
                g.wait()
            pcp.wait()

            @pl.loop(0, PC)
            def _(r):
                @plsc.parallel_loop(0, H, step=LANES, unroll=8)
                def _(c):
                    s = pl.ds(c, LANES)
                    pv = pos[k % 2].at[r, s][...]
                    for b in range(B):
                        ring[p].at[b * PC + r, s][...] = jnp.maximum(
                            ring[p].at[b * PC + r, s][...] + pv, 0.0)

            for b in range(B):
                out_cp[(k, b)] = pltpu.async_copy(
                    ring[p].at[pl.ds(b * PC, PC)],
                    o_hbm.at[pl.ds(b * L + l0 + k * PC, PC)],
                    so[p])

            if k + 2 < NCH:
                nxt = (k + 2) % 3
                if k - 1 >= 0:
                    for b in range(B):
                        out_cp.pop((k - 1, b)).wait()
                inflight[k + 2] = start(k + 2)

        for kk in (NCH - 2, NCH - 1):
            for b in range(B):
                out_cp.pop((kk, b)).wait()

    out = embed(W_word, W_pos, idx)
    return out.reshape(B, L, H)
